# baseline (speedup 1.0000x reference)
import functools

import jax
import jax.numpy as jnp
from jax import lax
from jax.experimental import pallas as pl
from jax.experimental.pallas import tpu as pltpu
from jax.experimental.pallas import tpu_sc as plsc

N = 10000
E = 320000
D = 128
H = 128
C = 47
CP = 128

NC = 2
NS = 16
NT = NC * NS
K = 80
NCHUNK = 128
E2 = NT * NCHUNK * K
CH0 = 64
CH1 = 2 * NCHUNK - CH0
NP = 10240
RPT = NP // NS
RB = K
NB = RPT // RB


def _mm_body(h_ref, w_ref, o_ref):
    o_ref[...] = jnp.dot(h_ref[...], w_ref[...],
                         precision=lax.Precision.HIGHEST,
                         preferred_element_type=jnp.float32)


def _matmul(h, w):
    n, d = h.shape
    m = w.shape[1]
    bn = 512
    return pl.pallas_call(
        _mm_body,
        grid=(pl.cdiv(n, bn),),
        in_specs=[pl.BlockSpec((bn, d), lambda i: (i, 0)),
                  pl.BlockSpec((d, m), lambda i: (0, 0))],
        out_specs=pl.BlockSpec((bn, m), lambda i: (i, 0)),
        out_shape=jax.ShapeDtypeStruct((n, m), jnp.float32),
    )(h, w)


def _combine_body(relu, h_ref, w_ref, b_ref, a0_ref, a1_ref, dp_ref, o_ref):
    deg = jnp.sum(dp_ref[...], axis=0)[:, None]
    inv = 1.0 / jnp.maximum(deg, 1.0)
    o = (jnp.dot(h_ref[...], w_ref[...],
                 precision=lax.Precision.HIGHEST,
                 preferred_element_type=jnp.float32)
         + b_ref[...] + (a0_ref[...] + a1_ref[...]) * inv)
    if relu:
        o = jnp.maximum(o, 0.0)
    o_ref[...] = o


def _combine(h, w, b, a0, a1, dp, relu):
    n, d = h.shape
    m = w.shape[1]
    bn = 512
    return pl.pallas_call(
        functools.partial(_combine_body, relu),
        grid=(pl.cdiv(n, bn),),
        in_specs=[pl.BlockSpec((bn, d), lambda i: (i, 0)),
                  pl.BlockSpec((d, m), lambda i: (0, 0)),
                  pl.BlockSpec((1, m), lambda i: (0, 0)),
                  pl.BlockSpec((bn, m), lambda i: (i, 0)),
                  pl.BlockSpec((bn, m), lambda i: (i, 0)),
                  pl.BlockSpec((NT, bn), lambda i: (0, i))],
        out_specs=pl.BlockSpec((bn, m), lambda i: (i, 0)),
        out_shape=jax.ShapeDtypeStruct((n, m), jnp.float32),
    )(h, w, b, a0, a1, dp)


def _make_sc_agg(w, with_deg):
    mesh = plsc.VectorSubcoreMesh(core_axis_name="c", subcore_axis_name="s")
    out_type = [jax.ShapeDtypeStruct((NC, NP, w), jnp.float32)]
    scratch = [
        pltpu.VMEM_SHARED((NP, w), jnp.float32),
        pltpu.VMEM((K,), jnp.int32),
        pltpu.VMEM((K,), jnp.int32),
        pltpu.VMEM((K,), jnp.int32),
        pltpu.VMEM((K,), jnp.int32),
        pltpu.VMEM((K, w), jnp.float32),
        pltpu.VMEM((K, w), jnp.float32),
        pltpu.VMEM((RB,), jnp.int32),
        pltpu.VMEM((RB,), jnp.int32),
        pltpu.SemaphoreType.DMA,
        pltpu.SemaphoreType.DMA,
    ]
    if with_deg:
        out_type.append(jax.ShapeDtypeStruct((NT, NP), jnp.float32))
        scratch.append(pltpu.VMEM((NP,), jnp.float32))

    @functools.partial(
        pl.kernel, out_type=out_type, mesh=mesh, scratch_types=scratch,
        compiler_params=pltpu.CompilerParams(needs_layout_passes=False))
    def sc_agg(*refs):
        if with_deg:
            (t_hbm, src_hbm, dst_hbm, z_hbm, zdeg_hbm,
             out_acc, out_deg,
             acc_sh, src0_v, src1_v, dst0_v, dst1_v, rows0_v, rows1_v,
             ix0_v, ix1_v, sem0, sem1, deg_v) = refs
        else:
            (t_hbm, src_hbm, dst_hbm, z_hbm,
             out_acc,
             acc_sh, src0_v, src1_v, dst0_v, dst1_v, rows0_v, rows1_v,
             ix0_v, ix1_v, sem0, sem1) = refs
        cid = lax.axis_index("c")
        sid = lax.axis_index("s")
        wid = sid * NC + cid
        row0 = sid * RPT
        iota = lax.iota(jnp.int32, 16)
        ones_lane = jnp.ones((16,), jnp.float32)
        pltpu.sync_copy(z_hbm, rows0_v)
        if with_deg:
            pltpu.sync_copy(zdeg_hbm, deg_v)

        def fill_ix(ix_v, r):
            for ii in range(RB // 16):
                ix_v[pl.ds(ii * 16, 16)] = iota + (r + ii * 16)

        def zero_chunk(i, carry):
            fill_ix(ix0_v, row0 + i * RB)
            pltpu.sync_copy(rows0_v, acc_sh.at[ix0_v])
            return carry

        lax.fori_loop(0, NB, zero_chunk, 0)
        plsc.subcore_barrier()

        def scatter_chunk(rows_v, dst_v):
            pltpu.sync_copy(rows_v, acc_sh.at[dst_v], add=True)
            if with_deg:
                for ii in range(K // 16):
                    idx = dst_v[pl.ds(ii * 16, 16)]
                    plsc.addupdate_scatter(deg_v, [idx], ones_lane)

        nch = jnp.where(cid == 0, CH0, CH1)
        base = jnp.where(cid == 0, sid * CH0, NS * CH0 + sid * CH1) * K

        def load_idx(src_v, dst_v, c):
            off = pl.multiple_of(base + c * K, 8)
            pltpu.sync_copy(src_hbm.at[pl.ds(off, K)], src_v)
            pltpu.sync_copy(dst_hbm.at[pl.ds(off, K)], dst_v)

        load_idx(src0_v, dst0_v, 0)
        pltpu.async_copy(t_hbm.at[src0_v], rows0_v, sem0)

        def pair(j, carry):
            c1 = 2 * j + 1
            c2 = jnp.minimum(2 * j + 2, nch - 1)
            load_idx(src1_v, dst1_v, c1)
            pltpu.make_async_copy(t_hbm.at[src0_v], rows0_v, sem0).wait()
            pltpu.async_copy(t_hbm.at[src1_v], rows1_v, sem1)
            scatter_chunk(rows0_v, dst0_v)
            load_idx(src0_v, dst0_v, c2)
            pltpu.make_async_copy(t_hbm.at[src1_v], rows1_v, sem1).wait()
            pltpu.async_copy(t_hbm.at[src0_v], rows0_v, sem0)
            scatter_chunk(rows1_v, dst1_v)
            return carry

        lax.fori_loop(0, nch // 2, pair, 0)
        pltpu.make_async_copy(t_hbm.at[src0_v], rows0_v, sem0).wait()

        plsc.subcore_barrier()

        fill_ix(ix0_v, row0)
        pltpu.async_copy(acc_sh.at[ix0_v], rows0_v, sem0)

        def read_pair(j, carry):
            c1 = 2 * j + 1
            c2 = jnp.minimum(2 * j + 2, NB - 1)
            fill_ix(ix1_v, row0 + c1 * RB)
            pltpu.make_async_copy(acc_sh.at[ix0_v], rows0_v, sem0).wait()
            pltpu.async_copy(acc_sh.at[ix1_v], rows1_v, sem1)
            r0 = pl.multiple_of(row0 + 2 * j * RB, 8)
            pltpu.sync_copy(rows0_v, out_acc.at[cid, pl.ds(r0, RB)])
            fill_ix(ix0_v, row0 + c2 * RB)
            pltpu.make_async_copy(acc_sh.at[ix1_v], rows1_v, sem1).wait()
            pltpu.async_copy(acc_sh.at[ix0_v], rows0_v, sem0)
            r1 = pl.multiple_of(row0 + c1 * RB, 8)
            pltpu.sync_copy(rows1_v, out_acc.at[cid, pl.ds(r1, RB)])
            return carry

        lax.fori_loop(0, NB // 2, read_pair, 0)
        pltpu.make_async_copy(acc_sh.at[ix0_v], rows0_v, sem0).wait()
        if with_deg:
            pltpu.sync_copy(deg_v, out_deg.at[wid])

    return sc_agg


_sc_agg_deg = _make_sc_agg(H, True)
_sc_agg_h = _make_sc_agg(H, False)


def kernel(x, edge_index, W_self0, W_neigh0, b0, W_self1, W_neigh1, b1,
           W_self2, W_neigh2, b2):
    src = jnp.concatenate([edge_index[0], jnp.zeros((E2 - E,), jnp.int32)])
    dst = jnp.concatenate(
        [edge_index[1], jnp.full((E2 - E,), NP - 1, jnp.int32)])
    zH = jnp.zeros((K, H), jnp.float32)
    zC = jnp.zeros((K, CP), jnp.float32)
    zdeg = jnp.zeros((NP,), jnp.float32)

    t0 = _matmul(x, W_neigh0)
    acc0, degp = _sc_agg_deg(t0, src, dst, zH, zdeg)
    dp = degp[:, :N]
    h1 = _combine(x, W_self0, b0.reshape(1, H), acc0[0, :N], acc0[1, :N],
                  dp, relu=True)

    t1 = _matmul(h1, W_neigh1)
    acc1 = _sc_agg_h(t1, src, dst, zH)[0]
    h2 = _combine(h1, W_self1, b1.reshape(1, H), acc1[0, :N], acc1[1, :N],
                  dp, relu=True)

    Wn2 = jnp.pad(W_neigh2, ((0, 0), (0, CP - C)))
    Ws2 = jnp.pad(W_self2, ((0, 0), (0, CP - C)))
    b2p = jnp.pad(b2, (0, CP - C)).reshape(1, CP)
    t2 = _matmul(h2, Wn2)
    acc2 = _sc_agg_h(t2, src, dst, zC)[0]
    out = _combine(h2, Ws2, b2p, acc2[0, :N], acc2[1, :N], dp, relu=False)
    return out[:, :C]


# speedup vs baseline: 1.2668x; 1.2668x over previous
import functools

import jax
import jax.numpy as jnp
from jax import lax
from jax.experimental import pallas as pl
from jax.experimental.pallas import tpu as pltpu
from jax.experimental.pallas import tpu_sc as plsc

N = 10000
E = 320000
D = 128
H = 128
C = 47
CP = 128

NC = 2
NS = 16
NT = NC * NS
K = 80
NCHUNK = 128
E2 = NT * NCHUNK * K
CH0 = 188
CH1 = 2 * NCHUNK - CH0
NP = 10240
RPT = NP // NS
RB = K
NB = RPT // RB


def _mm_body(h_ref, w_ref, o_ref):
    o_ref[...] = jnp.dot(h_ref[...], w_ref[...],
                         precision=lax.Precision.HIGHEST,
                         preferred_element_type=jnp.float32)


def _matmul(h, w):
    n, d = h.shape
    m = w.shape[1]
    bn = 512
    return pl.pallas_call(
        _mm_body,
        grid=(pl.cdiv(n, bn),),
        in_specs=[pl.BlockSpec((bn, d), lambda i: (i, 0)),
                  pl.BlockSpec((d, m), lambda i: (0, 0))],
        out_specs=pl.BlockSpec((bn, m), lambda i: (i, 0)),
        out_shape=jax.ShapeDtypeStruct((n, m), jnp.float32),
    )(h, w)


def _combine_body(relu, h_ref, w_ref, b_ref, a0_ref, a1_ref, dp_ref, o_ref):
    deg = jnp.sum(dp_ref[...], axis=0)[:, None]
    inv = 1.0 / jnp.maximum(deg, 1.0)
    o = (jnp.dot(h_ref[...], w_ref[...],
                 precision=lax.Precision.HIGHEST,
                 preferred_element_type=jnp.float32)
         + b_ref[...] + (a0_ref[...] + a1_ref[...]) * inv)
    if relu:
        o = jnp.maximum(o, 0.0)
    o_ref[...] = o


def _combine(h, w, b, a0, a1, dp, relu):
    n, d = h.shape
    m = w.shape[1]
    bn = 512
    return pl.pallas_call(
        functools.partial(_combine_body, relu),
        grid=(pl.cdiv(n, bn),),
        in_specs=[pl.BlockSpec((bn, d), lambda i: (i, 0)),
                  pl.BlockSpec((d, m), lambda i: (0, 0)),
                  pl.BlockSpec((1, m), lambda i: (0, 0)),
                  pl.BlockSpec((bn, m), lambda i: (i, 0)),
                  pl.BlockSpec((bn, m), lambda i: (i, 0)),
                  pl.BlockSpec((NT, bn), lambda i: (0, i))],
        out_specs=pl.BlockSpec((bn, m), lambda i: (i, 0)),
        out_shape=jax.ShapeDtypeStruct((n, m), jnp.float32),
    )(h, w, b, a0, a1, dp)


def _make_sc_agg(w, with_deg):
    mesh = plsc.VectorSubcoreMesh(core_axis_name="c", subcore_axis_name="s")
    out_type = [jax.ShapeDtypeStruct((NC, NP, w), jnp.float32)]
    scratch = [
        pltpu.VMEM_SHARED((NP, w), jnp.float32),
        pltpu.VMEM((K,), jnp.int32),
        pltpu.VMEM((K,), jnp.int32),
        pltpu.VMEM((K,), jnp.int32),
        pltpu.VMEM((K,), jnp.int32),
        pltpu.VMEM((K, w), jnp.float32),
        pltpu.VMEM((K, w), jnp.float32),
        pltpu.VMEM((RB,), jnp.int32),
        pltpu.VMEM((RB,), jnp.int32),
        pltpu.SemaphoreType.DMA,
        pltpu.SemaphoreType.DMA,
    ]
    if with_deg:
        out_type.append(jax.ShapeDtypeStruct((NT, NP), jnp.float32))
        scratch.append(pltpu.VMEM((NP,), jnp.float32))

    @functools.partial(
        pl.kernel, out_type=out_type, mesh=mesh, scratch_types=scratch,
        compiler_params=pltpu.CompilerParams(needs_layout_passes=False))
    def sc_agg(*refs):
        if with_deg:
            (t_hbm, src_hbm, dst_hbm, z_hbm, zdeg_hbm,
             out_acc, out_deg,
             acc_sh, src0_v, src1_v, dst0_v, dst1_v, rows0_v, rows1_v,
             ix0_v, ix1_v, sem0, sem1, deg_v) = refs
        else:
            (t_hbm, src_hbm, dst_hbm, z_hbm,
             out_acc,
             acc_sh, src0_v, src1_v, dst0_v, dst1_v, rows0_v, rows1_v,
             ix0_v, ix1_v, sem0, sem1) = refs
        cid = lax.axis_index("c")
        sid = lax.axis_index("s")
        wid = sid * NC + cid
        row0 = sid * RPT
        iota = lax.iota(jnp.int32, 16)
        ones_lane = jnp.ones((16,), jnp.float32)
        pltpu.sync_copy(z_hbm, rows0_v)
        if with_deg:
            pltpu.sync_copy(zdeg_hbm, deg_v)

        def fill_ix(ix_v, r):
            for ii in range(RB // 16):
                ix_v[pl.ds(ii * 16, 16)] = iota + (r + ii * 16)

        def zero_chunk(i, carry):
            fill_ix(ix0_v, row0 + i * RB)
            pltpu.sync_copy(rows0_v, acc_sh.at[ix0_v])
            return carry

        lax.fori_loop(0, NB, zero_chunk, 0)
        plsc.subcore_barrier()

        def scatter_chunk(rows_v, dst_v):
            pltpu.sync_copy(rows_v, acc_sh.at[dst_v], add=True)
            if with_deg:
                for ii in range(K // 16):
                    idx = dst_v[pl.ds(ii * 16, 16)]
                    plsc.addupdate_scatter(deg_v, [idx], ones_lane)

        nch = jnp.where(cid == 0, CH0, CH1)
        base = jnp.where(cid == 0, sid * CH0, NS * CH0 + sid * CH1) * K

        def load_idx(src_v, dst_v, c):
            off = pl.multiple_of(base + c * K, 8)
            pltpu.sync_copy(src_hbm.at[pl.ds(off, K)], src_v)
            pltpu.sync_copy(dst_hbm.at[pl.ds(off, K)], dst_v)

        load_idx(src0_v, dst0_v, 0)
        pltpu.async_copy(t_hbm.at[src0_v], rows0_v, sem0)

        def pair(j, carry):
            c1 = 2 * j + 1
            c2 = jnp.minimum(2 * j + 2, nch - 1)
            load_idx(src1_v, dst1_v, c1)
            pltpu.make_async_copy(t_hbm.at[src0_v], rows0_v, sem0).wait()
            pltpu.async_copy(t_hbm.at[src1_v], rows1_v, sem1)
            scatter_chunk(rows0_v, dst0_v)
            load_idx(src0_v, dst0_v, c2)
            pltpu.make_async_copy(t_hbm.at[src1_v], rows1_v, sem1).wait()
            pltpu.async_copy(t_hbm.at[src0_v], rows0_v, sem0)
            scatter_chunk(rows1_v, dst1_v)
            return carry

        lax.fori_loop(0, nch // 2, pair, 0)
        pltpu.make_async_copy(t_hbm.at[src0_v], rows0_v, sem0).wait()

        plsc.subcore_barrier()

        fill_ix(ix0_v, row0)
        pltpu.async_copy(acc_sh.at[ix0_v], rows0_v, sem0)

        def read_pair(j, carry):
            c1 = 2 * j + 1
            c2 = jnp.minimum(2 * j + 2, NB - 1)
            fill_ix(ix1_v, row0 + c1 * RB)
            pltpu.make_async_copy(acc_sh.at[ix0_v], rows0_v, sem0).wait()
            pltpu.async_copy(acc_sh.at[ix1_v], rows1_v, sem1)
            r0 = pl.multiple_of(row0 + 2 * j * RB, 8)
            pltpu.sync_copy(rows0_v, out_acc.at[cid, pl.ds(r0, RB)])
            fill_ix(ix0_v, row0 + c2 * RB)
            pltpu.make_async_copy(acc_sh.at[ix1_v], rows1_v, sem1).wait()
            pltpu.async_copy(acc_sh.at[ix0_v], rows0_v, sem0)
            r1 = pl.multiple_of(row0 + c1 * RB, 8)
            pltpu.sync_copy(rows1_v, out_acc.at[cid, pl.ds(r1, RB)])
            return carry

        lax.fori_loop(0, NB // 2, read_pair, 0)
        pltpu.make_async_copy(acc_sh.at[ix0_v], rows0_v, sem0).wait()
        if with_deg:
            pltpu.sync_copy(deg_v, out_deg.at[wid])

    return sc_agg


_sc_agg_deg = _make_sc_agg(H, True)
_sc_agg_h = _make_sc_agg(H, False)


def kernel(x, edge_index, W_self0, W_neigh0, b0, W_self1, W_neigh1, b1,
           W_self2, W_neigh2, b2):
    src = jnp.concatenate([edge_index[0], jnp.zeros((E2 - E,), jnp.int32)])
    dst = jnp.concatenate(
        [edge_index[1], jnp.full((E2 - E,), NP - 1, jnp.int32)])
    zH = jnp.zeros((K, H), jnp.float32)
    zC = jnp.zeros((K, CP), jnp.float32)
    zdeg = jnp.zeros((NP,), jnp.float32)

    t0 = _matmul(x, W_neigh0)
    acc0, degp = _sc_agg_deg(t0, src, dst, zH, zdeg)
    dp = degp[:, :N]
    h1 = _combine(x, W_self0, b0.reshape(1, H), acc0[0, :N], acc0[1, :N],
                  dp, relu=True)

    t1 = _matmul(h1, W_neigh1)
    acc1 = _sc_agg_h(t1, src, dst, zH)[0]
    h2 = _combine(h1, W_self1, b1.reshape(1, H), acc1[0, :N], acc1[1, :N],
                  dp, relu=True)

    Wn2 = jnp.pad(W_neigh2, ((0, 0), (0, CP - C)))
    Ws2 = jnp.pad(W_self2, ((0, 0), (0, CP - C)))
    b2p = jnp.pad(b2, (0, CP - C)).reshape(1, CP)
    t2 = _matmul(h2, Wn2)
    acc2 = _sc_agg_h(t2, src, dst, zC)[0]
    out = _combine(h2, Ws2, b2p, acc2[0, :N], acc2[1, :N], dp, relu=False)
    return out[:, :C]


# speedup vs baseline: 1.5814x; 1.2483x over previous
import functools

import jax
import jax.numpy as jnp
from jax import lax
from jax.experimental import pallas as pl
from jax.experimental.pallas import tpu as pltpu
from jax.experimental.pallas import tpu_sc as plsc

N = 10000
E = 320000
D = 128
H = 128
C = 47
CP = 128
W2 = 64

NC = 2
NS = 16
NT = NC * NS
K = 80
NCHUNK = 128
E2 = NT * NCHUNK * K
NP = 10240
RPT = NP // NS
RB = K
NB = RPT // RB


def _mm2_body(h_ref, w_ref, lo_ref, hi_ref):
    t = jnp.dot(h_ref[...], w_ref[...],
                precision=lax.Precision.HIGHEST,
                preferred_element_type=jnp.float32)
    lo_ref[...] = t[:, :W2]
    hi_ref[...] = t[:, W2:]


def _matmul2(h, w):
    d = h.shape[1]
    bn = 512
    return pl.pallas_call(
        _mm2_body,
        grid=(NP // bn,),
        in_specs=[pl.BlockSpec((bn, d), lambda i: (i, 0)),
                  pl.BlockSpec((d, 2 * W2), lambda i: (0, 0))],
        out_specs=[pl.BlockSpec((bn, W2), lambda i: (i, 0)),
                   pl.BlockSpec((bn, W2), lambda i: (i, 0))],
        out_shape=[jax.ShapeDtypeStruct((NP, W2), jnp.float32),
                   jax.ShapeDtypeStruct((NP, W2), jnp.float32)],
    )(h, w)


def _combine_body(relu, h_ref, w_ref, b_ref, a0l_ref, a0h_ref, a1l_ref,
                  a1h_ref, dp_ref, o_ref):
    deg = jnp.sum(dp_ref[...], axis=0)[:, None]
    inv = 1.0 / jnp.maximum(deg, 1.0)
    acc = jnp.concatenate(
        [a0l_ref[...] + a1l_ref[...], a0h_ref[...] + a1h_ref[...]], axis=1)
    o = (jnp.dot(h_ref[...], w_ref[...],
                 precision=lax.Precision.HIGHEST,
                 preferred_element_type=jnp.float32)
         + b_ref[...] + acc * inv)
    if relu:
        o = jnp.maximum(o, 0.0)
    o_ref[...] = o


def _combine(h, w, b, a0l, a0h, a1l, a1h, dp, relu):
    n, d = h.shape
    m = w.shape[1]
    bn = 512
    half = pl.BlockSpec((bn, W2), lambda i: (i, 0))
    return pl.pallas_call(
        functools.partial(_combine_body, relu),
        grid=(pl.cdiv(n, bn),),
        in_specs=[pl.BlockSpec((bn, d), lambda i: (i, 0)),
                  pl.BlockSpec((d, m), lambda i: (0, 0)),
                  pl.BlockSpec((1, m), lambda i: (0, 0)),
                  half, half, half, half,
                  pl.BlockSpec((NT, bn), lambda i: (0, i))],
        out_specs=pl.BlockSpec((bn, m), lambda i: (i, 0)),
        out_shape=jax.ShapeDtypeStruct((n, m), jnp.float32),
    )(h, w, b, a0l, a0h, a1l, a1h, dp)


def _make_sc_agg(with_deg):
    mesh = plsc.VectorSubcoreMesh(core_axis_name="c", subcore_axis_name="s")
    out_type = [jax.ShapeDtypeStruct((NC, NP, W2), jnp.float32),
                jax.ShapeDtypeStruct((NC, NP, W2), jnp.float32)]
    scratch = [
        pltpu.VMEM_SHARED((NP, W2), jnp.float32),
        pltpu.VMEM_SHARED((NP, W2), jnp.float32),
        pltpu.VMEM((K,), jnp.int32),
        pltpu.VMEM((K,), jnp.int32),
        pltpu.VMEM((K,), jnp.int32),
        pltpu.VMEM((K,), jnp.int32),
        pltpu.VMEM((K, W2), jnp.float32),
        pltpu.VMEM((K, W2), jnp.float32),
        pltpu.VMEM((RB, W2), jnp.float32),
        pltpu.VMEM((RB,), jnp.int32),
        pltpu.VMEM((RB,), jnp.int32),
        pltpu.SemaphoreType.DMA,
        pltpu.SemaphoreType.DMA,
    ]
    if with_deg:
        out_type.append(jax.ShapeDtypeStruct((NT, NP), jnp.float32))
        scratch.append(pltpu.VMEM((NP,), jnp.float32))

    @functools.partial(
        pl.kernel, out_type=out_type, mesh=mesh, scratch_types=scratch,
        compiler_params=pltpu.CompilerParams(needs_layout_passes=False))
    def sc_agg(*refs):
        if with_deg:
            (tlo_hbm, thi_hbm, src_hbm, dst_hbm, z_hbm, zdeg_hbm,
             out_lo, out_hi, out_deg,
             t_sh, acc_sh, src0_v, src1_v, dst0_v, dst1_v, rows0_v, rows1_v,
             stage_v, ix0_v, ix1_v, sem0, sem1, deg_v) = refs
        else:
            (tlo_hbm, thi_hbm, src_hbm, dst_hbm, z_hbm,
             out_lo, out_hi,
             t_sh, acc_sh, src0_v, src1_v, dst0_v, dst1_v, rows0_v, rows1_v,
             stage_v, ix0_v, ix1_v, sem0, sem1) = refs
        cid = lax.axis_index("c")
        sid = lax.axis_index("s")
        wid = sid * NC + cid
        row0 = sid * RPT
        iota = lax.iota(jnp.int32, 16)
        ones_lane = jnp.ones((16,), jnp.float32)
        if with_deg:
            pltpu.sync_copy(zdeg_hbm, deg_v)

        def fill_ix(ix_v, r):
            for ii in range(RB // 16):
                ix_v[pl.ds(ii * 16, 16)] = iota + (r + ii * 16)

        base = wid * NCHUNK * K

        def load_idx(src_v, dst_v, c):
            off = pl.multiple_of(base + c * K, 8)
            pltpu.sync_copy(src_hbm.at[pl.ds(off, K)], src_v)
            pltpu.sync_copy(dst_hbm.at[pl.ds(off, K)], dst_v)

        for p, (t_hbm, out_acc) in enumerate(
                [(tlo_hbm, out_lo), (thi_hbm, out_hi)]):
            pltpu.sync_copy(z_hbm, stage_v)

            def zero_chunk(i, carry):
                fill_ix(ix0_v, row0 + i * RB)
                pltpu.sync_copy(stage_v, acc_sh.at[ix0_v])
                return carry

            lax.fori_loop(0, NB, zero_chunk, 0)

            def stage_chunk(i, carry):
                r = row0 + i * RB
                fill_ix(ix0_v, r)
                pltpu.sync_copy(t_hbm.at[pl.ds(pl.multiple_of(r, 8), RB)],
                                stage_v)
                pltpu.sync_copy(stage_v, t_sh.at[ix0_v])
                return carry

            lax.fori_loop(0, NB, stage_chunk, 0)
            plsc.subcore_barrier()

            def scatter_chunk(rows_v, dst_v):
                pltpu.sync_copy(rows_v, acc_sh.at[dst_v], add=True)
                if with_deg and p == 0:
                    for ii in range(K // 16):
                        idx = dst_v[pl.ds(ii * 16, 16)]
                        plsc.addupdate_scatter(deg_v, [idx], ones_lane)

            load_idx(src0_v, dst0_v, 0)
            pltpu.async_copy(t_sh.at[src0_v], rows0_v, sem0)

            def pair(j, carry):
                c1 = 2 * j + 1
                c2 = jnp.minimum(2 * j + 2, NCHUNK - 1)
                load_idx(src1_v, dst1_v, c1)
                pltpu.make_async_copy(t_sh.at[src0_v], rows0_v, sem0).wait()
                pltpu.async_copy(t_sh.at[src1_v], rows1_v, sem1)
                scatter_chunk(rows0_v, dst0_v)
                load_idx(src0_v, dst0_v, c2)
                pltpu.make_async_copy(t_sh.at[src1_v], rows1_v, sem1).wait()
                pltpu.async_copy(t_sh.at[src0_v], rows0_v, sem0)
                scatter_chunk(rows1_v, dst1_v)
                return carry

            lax.fori_loop(0, NCHUNK // 2, pair, 0)
            pltpu.make_async_copy(t_sh.at[src0_v], rows0_v, sem0).wait()
            plsc.subcore_barrier()

            fill_ix(ix0_v, row0)
            pltpu.async_copy(acc_sh.at[ix0_v], rows0_v, sem0)

            def read_pair(j, carry):
                c1 = 2 * j + 1
                c2 = jnp.minimum(2 * j + 2, NB - 1)
                fill_ix(ix1_v, row0 + c1 * RB)
                pltpu.make_async_copy(acc_sh.at[ix0_v], rows0_v, sem0).wait()
                pltpu.async_copy(acc_sh.at[ix1_v], rows1_v, sem1)
                r0 = pl.multiple_of(row0 + 2 * j * RB, 8)
                pltpu.sync_copy(rows0_v, out_acc.at[cid, pl.ds(r0, RB)])
                fill_ix(ix0_v, row0 + c2 * RB)
                pltpu.make_async_copy(acc_sh.at[ix1_v], rows1_v, sem1).wait()
                pltpu.async_copy(acc_sh.at[ix0_v], rows0_v, sem0)
                r1 = pl.multiple_of(row0 + c1 * RB, 8)
                pltpu.sync_copy(rows1_v, out_acc.at[cid, pl.ds(r1, RB)])
                return carry

            lax.fori_loop(0, NB // 2, read_pair, 0)
            pltpu.make_async_copy(acc_sh.at[ix0_v], rows0_v, sem0).wait()
            plsc.subcore_barrier()

        if with_deg:
            pltpu.sync_copy(deg_v, out_deg.at[wid])

    return sc_agg


_sc_agg_deg = _make_sc_agg(True)
_sc_agg_h = _make_sc_agg(False)


def kernel(x, edge_index, W_self0, W_neigh0, b0, W_self1, W_neigh1, b1,
           W_self2, W_neigh2, b2):
    src = jnp.concatenate([edge_index[0], jnp.zeros((E2 - E,), jnp.int32)])
    dst = jnp.concatenate(
        [edge_index[1], jnp.full((E2 - E,), NP - 1, jnp.int32)])
    zW = jnp.zeros((RB, W2), jnp.float32)
    zdeg = jnp.zeros((NP,), jnp.float32)

    t0l, t0h = _matmul2(x, W_neigh0)
    a0l, a0h, degp = _sc_agg_deg(t0l, t0h, src, dst, zW, zdeg)
    dp = degp[:, :N]
    h1 = _combine(x, W_self0, b0.reshape(1, H), a0l[0, :N], a0h[0, :N],
                  a0l[1, :N], a0h[1, :N], dp, relu=True)

    t1l, t1h = _matmul2(h1, W_neigh1)
    a1l, a1h = _sc_agg_h(t1l, t1h, src, dst, zW)
    h2 = _combine(h1, W_self1, b1.reshape(1, H), a1l[0, :N], a1h[0, :N],
                  a1l[1, :N], a1h[1, :N], dp, relu=True)

    Wn2 = jnp.pad(W_neigh2, ((0, 0), (0, CP - C)))
    Ws2 = jnp.pad(W_self2, ((0, 0), (0, CP - C)))
    b2p = jnp.pad(b2, (0, CP - C)).reshape(1, CP)
    t2l, t2h = _matmul2(h2, Wn2)
    a2l, a2h = _sc_agg_h(t2l, t2h, src, dst, zW)
    out = _combine(h2, Ws2, b2p, a2l[0, :N], a2h[0, :N], a2l[1, :N],
                   a2h[1, :N], dp, relu=False)
    return out[:, :C]
